# baseline (speedup 1.0000x reference)
import functools

import jax
import jax.numpy as jnp
from jax import lax
from jax.experimental import pallas as pl
from jax.experimental.pallas import tpu as pltpu
from jax.experimental.pallas import tpu_sc as plsc

N = 10000
D = 128
H = 128
E = 320000
B = 4096

NPAD = 10240
BB = 512
KB = 2048

HALF = NPAD // 2
EPT = E // 16
ZROWS = NPAD // 32
ZBUF = 2 * NPAD
TRASH = N * NPAD
GROUPS = EPT // 16
INNER = 10


def _adj_body(rows_hbm, cols_hbm, a_ref, rows_v, cols_v, ones_v, zbuf, sem):
    cid = lax.axis_index("c")
    sid = lax.axis_index("s")

    pltpu.sync_copy(rows_hbm.at[pl.ds(sid * EPT, EPT)], rows_v)
    pltpu.sync_copy(cols_hbm.at[pl.ds(sid * EPT, EPT)], cols_v)

    ones_v[...] = jnp.ones((16,), jnp.float32)

    def zfill(i, carry):
        zbuf[pl.ds(i * 16, 16)] = jnp.zeros((16,), jnp.float32)
        return carry

    lax.fori_loop(0, ZBUF // 16, zfill, 0)

    rowbase = (cid * HALF + sid * ZROWS) * NPAD

    def zdma(k, carry):
        pltpu.sync_copy(zbuf, a_ref.at[pl.ds(rowbase + k * ZBUF, ZBUF)])
        return carry

    lax.fori_loop(0, ZROWS * NPAD // ZBUF, zdma, 0)

    plsc.subcore_barrier()

    lo = cid * HALF
    hi = lo + HALF

    def sgroup(i, carry):
        handles = []
        for t in range(INNER):
            off = (i * INNER + t) * 16
            r = rows_v[pl.ds(off, 16)]
            c = cols_v[pl.ds(off, 16)]
            flat = r * NPAD + c
            ok = (r >= lo) & (r < hi)
            idx = jnp.where(ok, flat, TRASH)
            handles.append(pltpu.async_copy(ones_v, a_ref.at[idx], sem))
        for h in handles:
            h.wait()
        return carry

    lax.fori_loop(0, GROUPS // INNER, sgroup, 0)


_adj_kernel = functools.partial(
    pl.kernel,
    out_type=jax.ShapeDtypeStruct((NPAD * NPAD,), jnp.float32),
    mesh=plsc.VectorSubcoreMesh(core_axis_name="c", subcore_axis_name="s"),
    scratch_types=[
        pltpu.VMEM((EPT,), jnp.int32),
        pltpu.VMEM((EPT,), jnp.int32),
        pltpu.VMEM((16,), jnp.float32),
        pltpu.VMEM((ZBUF,), jnp.float32),
        pltpu.SemaphoreType.DMA,
    ],
)(_adj_body)


def _spmm_body(ai_ref, aj_ref, x_ref, wxlinT_ref, bxlin_ref, out_ref):
    k = pl.program_id(1)

    @pl.when(k == 0)
    def _():
        out_ref[...] = jnp.zeros_like(out_ref)

    xb = x_ref[...]
    x2 = xb + jnp.dot(xb, wxlinT_ref[...], preferred_element_type=jnp.float32) \
            + bxlin_ref[...]
    cn = ai_ref[...] * aj_ref[...]
    out_ref[...] += jnp.dot(cn, x2, preferred_element_type=jnp.float32)


def _mlp_body(xi_ref, xj_ref, xcn_ref,
              wijiT_ref, biji_ref, wijjT_ref, bijj_ref, wijfT_ref, bijf_ref,
              wxcnT_ref, bxcn_ref, wxsT_ref, bxs_ref, beta_ref, sgn_ref,
              out_ref):
    xi = xi_ref[...]
    xj = xj_ref[...]
    xij = jnp.maximum(
        jnp.dot(xi, wijiT_ref[...], preferred_element_type=jnp.float32) + biji_ref[...]
        + jnp.dot(xj, wijjT_ref[...], preferred_element_type=jnp.float32) + bijj_ref[...],
        0.0)
    xij = jnp.dot(xij, wijfT_ref[...], preferred_element_type=jnp.float32) + bijf_ref[...]
    xs = (jnp.dot(xcn_ref[...], wxcnT_ref[...], preferred_element_type=jnp.float32)
          + bxcn_ref[...]) * beta_ref[0, 0] + xij
    xs = jnp.dot(xs, wxsT_ref[...], preferred_element_type=jnp.float32) + bxs_ref[...]
    z = sgn_ref[0, 0] * xs
    t = -z
    out_ref[...] = jnp.maximum(t, 0.0) + jnp.log1p(jnp.exp(-jnp.abs(t)))


@jax.jit
def _run(x, edge_index, tar_ei, boolen, beta, Wxlin, bxlin, Wxcn, bxcn,
         Wiji, biji, Wijj, bijj, Wijf, bijf, Wxs, bxs):
    a_flat = _adj_kernel(edge_index[0], edge_index[1])
    a2 = a_flat.reshape(NPAD, NPAD)
    ai = jnp.take(a2, tar_ei[0], axis=0)
    aj = jnp.take(a2, tar_ei[1], axis=0)
    xpad = jnp.pad(x, ((0, NPAD - N), (0, 0)))

    xcn = pl.pallas_call(
        _spmm_body,
        grid=(B // BB, NPAD // KB),
        in_specs=[
            pl.BlockSpec((BB, KB), lambda i, k: (i, k)),
            pl.BlockSpec((BB, KB), lambda i, k: (i, k)),
            pl.BlockSpec((KB, D), lambda i, k: (k, 0)),
            pl.BlockSpec((H, H), lambda i, k: (0, 0)),
            pl.BlockSpec((1, H), lambda i, k: (0, 0)),
        ],
        out_specs=pl.BlockSpec((BB, D), lambda i, k: (i, 0)),
        out_shape=jax.ShapeDtypeStruct((B, D), jnp.float32),
        compiler_params=pltpu.CompilerParams(
            dimension_semantics=("parallel", "arbitrary")),
    )(ai, aj, xpad, Wxlin.T, bxlin.reshape(1, H))

    xi = jnp.take(x, tar_ei[0], axis=0)
    xj = jnp.take(x, tar_ei[1], axis=0)
    sgn = jnp.where(boolen, 1.0, -1.0).reshape(1, 1).astype(jnp.float32)

    res = pl.pallas_call(
        _mlp_body,
        grid=(B // BB,),
        in_specs=[
            pl.BlockSpec((BB, D), lambda i: (i, 0)),
            pl.BlockSpec((BB, D), lambda i: (i, 0)),
            pl.BlockSpec((BB, D), lambda i: (i, 0)),
            pl.BlockSpec((D, H), lambda i: (0, 0)),
            pl.BlockSpec((1, H), lambda i: (0, 0)),
            pl.BlockSpec((D, H), lambda i: (0, 0)),
            pl.BlockSpec((1, H), lambda i: (0, 0)),
            pl.BlockSpec((H, H), lambda i: (0, 0)),
            pl.BlockSpec((1, H), lambda i: (0, 0)),
            pl.BlockSpec((D, H), lambda i: (0, 0)),
            pl.BlockSpec((1, H), lambda i: (0, 0)),
            pl.BlockSpec((H, 1), lambda i: (0, 0)),
            pl.BlockSpec((1, 1), lambda i: (0, 0)),
            pl.BlockSpec((1, 1), lambda i: (0, 0)),
            pl.BlockSpec((1, 1), lambda i: (0, 0)),
        ],
        out_specs=pl.BlockSpec((BB, 1), lambda i: (i, 0)),
        out_shape=jax.ShapeDtypeStruct((B, 1), jnp.float32),
    )(xi, xj, xcn,
      Wiji.T, biji.reshape(1, H), Wijj.T, bijj.reshape(1, H),
      Wijf.T, bijf.reshape(1, H), Wxcn.T, bxcn.reshape(1, H),
      Wxs.T, bxs.reshape(1, 1), beta.reshape(1, 1), sgn)
    return res


def kernel(x, edge_index, tar_ei, boolen, beta, Wxlin, bxlin, Wxcn, bxcn,
           Wiji, biji, Wijj, bijj, Wijf, bijf, Wxs, bxs):
    return _run(x, edge_index, tar_ei, boolen, beta, Wxlin, bxlin, Wxcn, bxcn,
                Wiji, biji, Wijj, bijj, Wijf, bijf, Wxs, bxs)
